# baseline (speedup 1.0000x reference)
import functools

import jax
import jax.numpy as jnp
from jax import lax
from jax.experimental import pallas as pl
from jax.experimental.pallas import tpu as pltpu
from jax.experimental.pallas import tpu_sc as plsc

D = 64
BR = 64
NBUF = 2

_info = plsc.get_sparse_core_info()
NW = _info.num_cores * _info.num_subcores
L = _info.num_lanes


def _sc_body(table_hbm, labt_hbm, out_hbm, table_v, idx_a, idx_b, buf_a,
             buf_b, ssem_a, ssem_b, *, n_rows, n_labels):
    wid = lax.axis_index("s") * _info.num_cores + lax.axis_index("c")
    rows_per_w = n_rows // NW
    nblocks = rows_per_w // BR
    r0w = wid * rows_per_w

    pltpu.sync_copy(table_hbm, table_v)

    def do_block(b, idx_v, buf, ssem):
        r0 = r0w + b * BR
        s = r0 // (8 * n_labels // 16)
        d_hi = (r0 // 1024) % 8
        nbase = (r0 // 8) % 128 * 128
        pltpu.sync_copy(labt_hbm.at[s, pl.ds(nbase, 8 * 128)], idx_v)

        def inner(nl, _):
            idx = [idx_v[pl.ds(nl * 128 + j * L, L)] for j in range(8)]
            for d_lo in range(8):
                d = d_hi * 8 + d_lo
                dvec = jnp.full((L,), d, dtype=jnp.int32)
                for j in range(8):
                    v = plsc.load_gather(table_v, [idx[j], dvec])
                    buf[nl * 8 + d_lo, pl.ds(j * L, L)] = v
            return 0

        lax.fori_loop(0, 8, inner, 0)
        pltpu.async_copy(buf, out_hbm.at[pl.ds(r0, BR)], ssem)

    def wait_block(b, buf, ssem):
        r0 = r0w + b * BR
        pltpu.make_async_copy(buf, out_hbm.at[pl.ds(r0, BR)], ssem).wait()

    do_block(0, idx_a, buf_a, ssem_a)

    def body(i, _):
        b0 = 2 * i
        do_block(b0 + 1, idx_b, buf_b, ssem_b)
        wait_block(b0, buf_a, ssem_a)

        @pl.when(i < nblocks // 2 - 1)
        def _():
            do_block(b0 + 2, idx_a, buf_a, ssem_a)

        wait_block(b0 + 1, buf_b, ssem_b)
        return 0

    lax.fori_loop(0, nblocks // 2, body, 0)


def kernel(labels, embeddings):
    n, s = labels.shape
    total = n * s
    n_rows = total * D // 128
    assert n_rows % (NW * BR) == 0 and n % 2048 == 0

    labt = labels.astype(jnp.int32).T

    mesh = plsc.VectorSubcoreMesh(core_axis_name="c", subcore_axis_name="s")
    k = functools.partial(
        pl.kernel,
        mesh=mesh,
        out_type=jax.ShapeDtypeStruct((n_rows, 128), jnp.float32),
        scratch_types=[
            pltpu.VMEM((1000, D), jnp.float32),
            pltpu.VMEM((8 * 128,), jnp.int32),
            pltpu.VMEM((8 * 128,), jnp.int32),
            pltpu.VMEM((BR, 128), jnp.float32),
            pltpu.VMEM((BR, 128), jnp.float32),
            pltpu.SemaphoreType.DMA,
            pltpu.SemaphoreType.DMA,
        ],
        compiler_params=pltpu.CompilerParams(use_tc_tiling_on_sc=False,
                                             needs_layout_passes=False),
    )(functools.partial(_sc_body, n_rows=n_rows, n_labels=n))
    x = k(embeddings, labt)
    return (x.reshape(s, 8, n // 128, 8, 128)
            .transpose(2, 4, 0, 1, 3).reshape(n, s, D))


# speedup vs baseline: 1.2267x; 1.2267x over previous
import functools

import jax
import jax.numpy as jnp
from jax import lax
from jax.experimental import pallas as pl
from jax.experimental.pallas import tpu as pltpu
from jax.experimental.pallas import tpu_sc as plsc

D = 64
BR = 64
NBUF = 2

_info = plsc.get_sparse_core_info()
NW = _info.num_cores * _info.num_subcores
L = _info.num_lanes


def _sc_body(table_hbm, labt_hbm, out_hbm, table_v, idx_a, idx_b, buf_a,
             buf_b, ssem_a, ssem_b, *, n_rows, n_labels):
    wid = lax.axis_index("s") * _info.num_cores + lax.axis_index("c")
    rows_per_w = n_rows // NW
    nblocks = rows_per_w // BR
    r0w = wid * rows_per_w

    pltpu.sync_copy(table_hbm, table_v)

    def do_block(b, idx_v, buf, ssem):
        r0 = r0w + b * BR
        s = r0 // (8 * n_labels // 16)
        d_hi = (r0 // 1024) % 8
        nbase = (r0 // 8) % 128 * 128
        pltpu.sync_copy(labt_hbm.at[s, pl.ds(nbase, 8 * 128)], idx_v)

        def inner(nl, _):
            idx = [idx_v[pl.ds(nl * 128 + j * L, L)] for j in range(8)]
            for d_lo in range(8):
                d = d_hi * 8 + d_lo
                vals = [plsc.load_gather(table_v, [idx[j] + d])
                        for j in range(8)]
                for j in range(8):
                    buf[nl * 8 + d_lo, pl.ds(j * L, L)] = vals[j]
            return 0

        lax.fori_loop(0, 8, inner, 0)
        pltpu.async_copy(buf, out_hbm.at[pl.ds(r0, BR)], ssem)

    def wait_block(b, buf, ssem):
        r0 = r0w + b * BR
        pltpu.make_async_copy(buf, out_hbm.at[pl.ds(r0, BR)], ssem).wait()

    do_block(0, idx_a, buf_a, ssem_a)

    def body(i, _):
        b0 = 2 * i
        do_block(b0 + 1, idx_b, buf_b, ssem_b)
        wait_block(b0, buf_a, ssem_a)

        @pl.when(i < nblocks // 2 - 1)
        def _():
            do_block(b0 + 2, idx_a, buf_a, ssem_a)

        wait_block(b0 + 1, buf_b, ssem_b)
        return 0

    lax.fori_loop(0, nblocks // 2, body, 0)


def kernel(labels, embeddings):
    n, s = labels.shape
    total = n * s
    n_rows = total * D // 128
    assert n_rows % (NW * BR) == 0 and n % 2048 == 0

    labt = labels.astype(jnp.int32).T * D
    table_flat = embeddings.reshape(-1)

    mesh = plsc.VectorSubcoreMesh(core_axis_name="c", subcore_axis_name="s")
    k = functools.partial(
        pl.kernel,
        mesh=mesh,
        out_type=jax.ShapeDtypeStruct((n_rows, 128), jnp.float32),
        scratch_types=[
            pltpu.VMEM((1000 * D,), jnp.float32),
            pltpu.VMEM((8 * 128,), jnp.int32),
            pltpu.VMEM((8 * 128,), jnp.int32),
            pltpu.VMEM((BR, 128), jnp.float32),
            pltpu.VMEM((BR, 128), jnp.float32),
            pltpu.SemaphoreType.DMA,
            pltpu.SemaphoreType.DMA,
        ],
        compiler_params=pltpu.CompilerParams(use_tc_tiling_on_sc=False,
                                             needs_layout_passes=False),
    )(functools.partial(_sc_body, n_rows=n_rows, n_labels=n))
    x = k(table_flat, labt)
    return (x.reshape(s, 8, n // 128, 8, 128)
            .transpose(2, 4, 0, 1, 3).reshape(n, s, D))


# speedup vs baseline: 2.1728x; 1.7712x over previous
import functools

import jax
import jax.numpy as jnp
from jax import lax
from jax.experimental import pallas as pl
from jax.experimental.pallas import tpu as pltpu
from jax.experimental.pallas import tpu_sc as plsc

D = 64
S = 50
LR = 16

_info = plsc.get_sparse_core_info()
NW = _info.num_cores * _info.num_subcores


def _sc_gather(table_hbm, labels_hbm, out_hbm, table_v, idx_a, idx_b, buf_a,
               buf_b, gsem_a, gsem_b, ssem_a, ssem_b, *, n_groups):
    wid = lax.axis_index("s") * _info.num_cores + lax.axis_index("c")
    wrow = wid * (n_groups * LR)

    @pl.when(lax.axis_index("s") == 0)
    def _():
        pltpu.sync_copy(table_hbm, table_v)

    plsc.subcore_barrier()

    def fire_group(g, idx_v, buf, gsem):
        pltpu.sync_copy(labels_hbm.at[pl.ds(wrow + g * LR, LR)], idx_v)
        for r in range(LR):
            pltpu.async_copy(table_v.at[idx_v.at[r]], buf.at[r], gsem)

    def wait_gathers(buf, gsem):
        pltpu.make_async_copy(out_hbm.at[pl.ds(0, LR)], buf, gsem).wait()

    def fire_store(g, buf, ssem):
        pltpu.async_copy(buf, out_hbm.at[pl.ds(wrow + g * LR, LR)], ssem)

    def wait_store(g, buf, ssem):
        pltpu.make_async_copy(
            buf, out_hbm.at[pl.ds(wrow + g * LR, LR)], ssem).wait()

    fire_group(0, idx_a, buf_a, gsem_a)

    def body(i, _):
        g0 = 2 * i
        g1 = g0 + 1

        @pl.when(i > 0)
        def _():
            wait_store(g0 - 1, buf_b, ssem_b)

        fire_group(g1, idx_b, buf_b, gsem_b)
        wait_gathers(buf_a, gsem_a)
        fire_store(g0, buf_a, ssem_a)

        @pl.when(i < (n_groups // 2 - 1))
        def _():
            wait_store(g0, buf_a, ssem_a)
            fire_group(g0 + 2, idx_a, buf_a, gsem_a)

        wait_gathers(buf_b, gsem_b)
        fire_store(g1, buf_b, ssem_b)
        return 0

    lax.fori_loop(0, n_groups // 2, body, 0)
    wait_store(n_groups - 2, buf_a, ssem_a)
    wait_store(n_groups - 1, buf_b, ssem_b)


def kernel(labels, embeddings):
    n, s = labels.shape
    assert s == S and n % (NW * LR) == 0
    n_groups = n // (NW * LR)

    idx = labels.astype(jnp.int32)

    mesh = plsc.VectorSubcoreMesh(core_axis_name="c", subcore_axis_name="s")
    k = functools.partial(
        pl.kernel,
        mesh=mesh,
        out_type=jax.ShapeDtypeStruct((n, S, D), jnp.float32),
        scratch_types=[
            pltpu.VMEM_SHARED((1000, D), jnp.float32),
            pltpu.VMEM((LR, S), jnp.int32),
            pltpu.VMEM((LR, S), jnp.int32),
            pltpu.VMEM((LR, S, D), jnp.float32),
            pltpu.VMEM((LR, S, D), jnp.float32),
            pltpu.SemaphoreType.DMA,
            pltpu.SemaphoreType.DMA,
            pltpu.SemaphoreType.DMA,
            pltpu.SemaphoreType.DMA,
        ],
        compiler_params=pltpu.CompilerParams(use_tc_tiling_on_sc=False),
    )(functools.partial(_sc_gather, n_groups=n_groups))
    return k(embeddings, idx)
